# baseline (speedup 1.0000x reference)
import jax
import jax.numpy as jnp
from jax import lax
from jax.experimental import pallas as pl
from jax.experimental.pallas import tpu as pltpu
from jax.experimental.pallas import tpu_sc as plsc

_B = 16384
_L = 20
_D = 128
_NC = 2
_NS = 16
_NW = _NC * _NS
_BPW = _B // _NW
_CH = 128
_NCHUNK = _BPW // _CH
_IC = _CH * _L

_LANES = 16


def _sc_pool_body(idx_hbm, tab_hbm, out_hbm,
                  idx0_v, idx1_v, acc_v,
                  gsem0, gsem1, isem0, isem1):
    wid = lax.axis_index("s") * _NC + lax.axis_index("c")
    base = wid * _BPW
    idx_bufs = (idx0_v, idx1_v)
    tr_bufs = idx_bufs
    isems = (isem0, isem1)
    gsems = (gsem0, gsem1)

    def start_idx(g, slot):
        off = (base + g * _CH) * _L
        pltpu.async_copy(idx_hbm.at[pl.ds(off, _IC)],
                         idx_bufs[slot], isems[slot])

    def wait_idx(slot):
        pltpu.make_async_copy(idx_hbm.at[pl.ds(0, _IC)],
                              idx_bufs[slot], isems[slot]).wait()

    def start_adds(slot):
        for l in range(_L):
            pltpu.async_copy(
                tab_hbm.at[tr_bufs[slot].at[pl.ds(l * _CH, _CH)]],
                acc_v.at[slot], gsems[slot], add=True)

    def wait_adds(slot):
        for l in range(_L):
            pltpu.make_async_copy(
                tab_hbm.at[tr_bufs[slot].at[pl.ds(l * _CH, _CH)]],
                acc_v.at[slot], gsems[slot]).wait()

    def zero_acc(slot):
        z = jnp.zeros((_LANES,), jnp.float32)

        def b_body(b, _):
            for c in range(_D // _LANES):
                acc_v[slot, b, pl.ds(c * _LANES, _LANES)] = z
            return 0

        lax.fori_loop(0, _CH, b_body, 0)

    def copy_out(g, slot):
        pltpu.sync_copy(acc_v.at[slot], out_hbm.at[pl.ds(base + g * _CH, _CH)])

    zero_acc(0)
    zero_acc(1)
    start_idx(0, 0)
    start_idx(1, 1)
    wait_idx(0)
    start_adds(0)

    def step(g, slot, last=False):
        other = 1 - slot
        wait_idx(other)
        start_adds(other)
        wait_adds(slot)
        if not last:
            start_idx(g + 2, slot)
        copy_out(g, slot)
        if not last:
            zero_acc(slot)

    def g2_body(g2, _):
        a = 2 * g2
        step(a, 0)
        step(a + 1, 1)
        return 0

    lax.fori_loop(0, _NCHUNK // 2 - 1, g2_body, 0)
    a = _NCHUNK - 2
    wait_idx(1)
    start_adds(1)
    wait_adds(0)
    copy_out(a, 0)
    wait_adds(1)
    copy_out(a + 1, 1)


_sc_pool = pl.kernel(
    _sc_pool_body,
    out_type=jax.ShapeDtypeStruct((_B, _D), jnp.float32),
    mesh=plsc.VectorSubcoreMesh(core_axis_name="c", subcore_axis_name="s"),
    scratch_types=[
        pltpu.VMEM((_IC,), jnp.int32),
        pltpu.VMEM((_IC,), jnp.int32),
        pltpu.VMEM((2, _CH, _D), jnp.float32),
        pltpu.SemaphoreType.DMA,
        pltpu.SemaphoreType.DMA,
        pltpu.SemaphoreType.DMA,
        pltpu.SemaphoreType.DMA,
    ],
)


_MBLK = 2048


def _one_tower(x, w0, b0, w1, b1, w2, b2):
    dn = (((1,), (1,)), ((), ()))
    h = jnp.maximum(lax.dot_general(
        x, w0, dn, preferred_element_type=jnp.float32) + b0, 0.0)
    h = jnp.maximum(lax.dot_general(
        h, w1, dn, preferred_element_type=jnp.float32) + b1, 0.0)
    return jnp.maximum(lax.dot_general(
        h, w2, dn, preferred_element_type=jnp.float32) + b2, 0.0)


def _mlp_body(qx_ref, cx_ref,
              qw0_ref, qb0_ref, qw1_ref, qb1_ref, qw2_ref, qb2_ref,
              cw0_ref, cb0_ref, cw1_ref, cb1_ref, cw2_ref, cb2_ref,
              qo_ref, co_ref):
    qo_ref[...] = _one_tower(qx_ref[...], qw0_ref[...], qb0_ref[...],
                             qw1_ref[...], qb1_ref[...],
                             qw2_ref[...], qb2_ref[...])
    co_ref[...] = _one_tower(cx_ref[...], cw0_ref[...], cb0_ref[...],
                             cw1_ref[...], cb1_ref[...],
                             cw2_ref[...], cb2_ref[...])


def _mlp_towers(q_pooled, c_pooled, qws, cws):
    n0, n1, n2 = 128, 64, 32
    x_spec = pl.BlockSpec((_MBLK, _D), lambda i: (i, 0))
    full = lambda s: pl.BlockSpec(s, lambda i: tuple(0 for _ in s))
    w_specs = [full((n0, _D)), full((1, n0)),
               full((n1, n0)), full((1, n1)),
               full((n2, n1)), full((1, n2))]
    return pl.pallas_call(
        _mlp_body,
        grid=(_B // _MBLK,),
        in_specs=[x_spec, x_spec] + w_specs + w_specs,
        out_specs=[pl.BlockSpec((_MBLK, n2), lambda i: (i, 0))] * 2,
        out_shape=[jax.ShapeDtypeStruct((_B, n2), jnp.float32)] * 2,
    )(q_pooled, c_pooled, *qws, *cws)


def _permute_indices(idx):
    return (idx.astype(jnp.int32)
            .reshape(_NW, _NCHUNK, _CH, _L)
            .transpose(0, 1, 3, 2)
            .reshape(-1))


def kernel(query_indices, candidate_indices, q_table, c_table,
           q_w0, q_b0, q_w1, q_b1, q_w2, q_b2,
           c_w0, c_b0, c_w1, c_b1, c_w2, c_b2):
    qi = _permute_indices(query_indices)
    ci = _permute_indices(candidate_indices)
    qws = (q_w0, q_b0.reshape(1, -1), q_w1, q_b1.reshape(1, -1),
           q_w2, q_b2.reshape(1, -1))
    cws = (c_w0, c_b0.reshape(1, -1), c_w1, c_b1.reshape(1, -1),
           c_w2, c_b2.reshape(1, -1))
    q_pooled = _sc_pool(qi, q_table)
    c_pooled = _sc_pool(ci, c_table)
    return _mlp_towers(q_pooled, c_pooled, qws, cws)


# speedup vs baseline: 1.0340x; 1.0340x over previous
import jax
import jax.numpy as jnp
from jax import lax
from jax.experimental import pallas as pl
from jax.experimental.pallas import tpu as pltpu
from jax.experimental.pallas import tpu_sc as plsc

_B = 16384
_L = 20
_D = 128
_NC = 2
_NS = 16
_NW = _NC * _NS
_BPW = _B // _NW
_CH = 128
_NCHUNK = _BPW // _CH
_IC = _CH * _L

_LANES = 16


def _sc_pool_body(idx_hbm, tab_hbm, out_hbm,
                  idx0_v, idx1_v, acc_v,
                  gsem0, gsem1, isem0, isem1):
    wid = lax.axis_index("s") * _NC + lax.axis_index("c")
    base = wid * _BPW
    idx_bufs = (idx0_v, idx1_v)
    tr_bufs = idx_bufs
    isems = (isem0, isem1)
    gsems = (gsem0, gsem1)

    def start_idx(g, slot):
        off = (base + g * _CH) * _L
        pltpu.async_copy(idx_hbm.at[pl.ds(off, _IC)],
                         idx_bufs[slot], isems[slot])

    def wait_idx(slot):
        pltpu.make_async_copy(idx_hbm.at[pl.ds(0, _IC)],
                              idx_bufs[slot], isems[slot]).wait()

    def start_adds(slot):
        for l in range(_L):
            pltpu.async_copy(
                tab_hbm.at[tr_bufs[slot].at[pl.ds(l * _CH, _CH)]],
                acc_v.at[slot], gsems[slot], add=True)

    def wait_adds(slot):
        for l in range(_L):
            pltpu.make_async_copy(
                tab_hbm.at[tr_bufs[slot].at[pl.ds(l * _CH, _CH)]],
                acc_v.at[slot], gsems[slot]).wait()

    def zero_acc(slot):
        z = jnp.zeros((_LANES,), jnp.float32)

        def b_body(b, _):
            for c in range(_D // _LANES):
                acc_v[slot, b, pl.ds(c * _LANES, _LANES)] = z
            return 0

        lax.fori_loop(0, _CH, b_body, 0)

    def copy_out(g, slot):
        pltpu.sync_copy(acc_v.at[slot], out_hbm.at[pl.ds(base + g * _CH, _CH)])

    zero_acc(0)
    zero_acc(1)
    start_idx(0, 0)
    start_idx(1, 1)
    wait_idx(0)
    start_adds(0)

    def step(g, slot, last=False):
        other = 1 - slot
        wait_idx(other)
        start_adds(other)
        wait_adds(slot)
        if not last:
            start_idx(g + 2, slot)
        copy_out(g, slot)
        if not last:
            zero_acc(slot)

    def g2_body(g2, _):
        a = 2 * g2
        step(a, 0)
        step(a + 1, 1)
        return 0

    lax.fori_loop(0, _NCHUNK // 2 - 1, g2_body, 0)
    a = _NCHUNK - 2
    wait_idx(1)
    start_adds(1)
    wait_adds(0)
    copy_out(a, 0)
    wait_adds(1)
    copy_out(a + 1, 1)


_sc_pool = pl.kernel(
    _sc_pool_body,
    out_type=jax.ShapeDtypeStruct((_B, _D), jnp.float32),
    mesh=plsc.VectorSubcoreMesh(core_axis_name="c", subcore_axis_name="s"),
    scratch_types=[
        pltpu.VMEM((_IC,), jnp.int32),
        pltpu.VMEM((_IC,), jnp.int32),
        pltpu.VMEM((2, _CH, _D), jnp.float32),
        pltpu.SemaphoreType.DMA,
        pltpu.SemaphoreType.DMA,
        pltpu.SemaphoreType.DMA,
        pltpu.SemaphoreType.DMA,
    ],
)


_MBLK = 4096


def _one_tower(x, w0, b0, w1, b1, w2, b2):
    dn = (((1,), (1,)), ((), ()))
    h = jnp.maximum(lax.dot_general(
        x, w0, dn, preferred_element_type=jnp.float32) + b0, 0.0)
    h = jnp.maximum(lax.dot_general(
        h, w1, dn, preferred_element_type=jnp.float32) + b1, 0.0)
    return jnp.maximum(lax.dot_general(
        h, w2, dn, preferred_element_type=jnp.float32) + b2, 0.0)


def _mlp_body(qx_ref, cx_ref,
              qw0_ref, qb0_ref, qw1_ref, qb1_ref, qw2_ref, qb2_ref,
              cw0_ref, cb0_ref, cw1_ref, cb1_ref, cw2_ref, cb2_ref,
              qo_ref, co_ref):
    qo_ref[...] = _one_tower(qx_ref[...], qw0_ref[...], qb0_ref[...],
                             qw1_ref[...], qb1_ref[...],
                             qw2_ref[...], qb2_ref[...])
    co_ref[...] = _one_tower(cx_ref[...], cw0_ref[...], cb0_ref[...],
                             cw1_ref[...], cb1_ref[...],
                             cw2_ref[...], cb2_ref[...])


def _mlp_tower(pooled, ws):
    n0, n1, n2 = 128, 64, 32
    x_spec = pl.BlockSpec((_MBLK, _D), lambda i: (i, 0))
    full = lambda s: pl.BlockSpec(s, lambda i: tuple(0 for _ in s))
    w_specs = [full((n0, _D)), full((1, n0)),
               full((n1, n0)), full((1, n1)),
               full((n2, n1)), full((1, n2))]
    return pl.pallas_call(
        lambda x_ref, w0, b0, w1, b1, w2, b2, o_ref: o_ref.__setitem__(
            ..., _one_tower(x_ref[...], w0[...], b0[...], w1[...], b1[...],
                            w2[...], b2[...])),
        grid=(_B // _MBLK,),
        in_specs=[x_spec] + w_specs,
        out_specs=pl.BlockSpec((_MBLK, n2), lambda i: (i, 0)),
        out_shape=jax.ShapeDtypeStruct((_B, n2), jnp.float32),
    )(pooled, *ws)


def _permute_indices(idx):
    return (idx.astype(jnp.int32)
            .reshape(_NW, _NCHUNK, _CH, _L)
            .transpose(0, 1, 3, 2)
            .reshape(-1))


def kernel(query_indices, candidate_indices, q_table, c_table,
           q_w0, q_b0, q_w1, q_b1, q_w2, q_b2,
           c_w0, c_b0, c_w1, c_b1, c_w2, c_b2):
    qi = _permute_indices(query_indices)
    ci = _permute_indices(candidate_indices)
    qws = (q_w0, q_b0.reshape(1, -1), q_w1, q_b1.reshape(1, -1),
           q_w2, q_b2.reshape(1, -1))
    cws = (c_w0, c_b0.reshape(1, -1), c_w1, c_b1.reshape(1, -1),
           c_w2, c_b2.reshape(1, -1))
    q_pooled = _sc_pool(qi, q_table)
    c_pooled = _sc_pool(ci, c_table)
    query_embedding = _mlp_tower(q_pooled, qws)
    candidate_embedding = _mlp_tower(c_pooled, cws)
    return (query_embedding, candidate_embedding)


# speedup vs baseline: 1.0449x; 1.0106x over previous
import jax
import jax.numpy as jnp
from jax import lax
from jax.experimental import pallas as pl
from jax.experimental.pallas import tpu as pltpu
from jax.experimental.pallas import tpu_sc as plsc

_B = 16384
_L = 20
_D = 128
_NC = 2
_NS = 16
_NW = _NC * _NS
_BPW = _B // _NW
_CH = 128
_NCHUNK = _BPW // _CH
_IC = _CH * _L

_LANES = 16


def _sc_pool_body(idx_hbm, tab_hbm, out_hbm,
                  idx0_v, idx1_v, acc_v,
                  gsem0, gsem1, isem0, isem1):
    wid = lax.axis_index("s") * _NC + lax.axis_index("c")
    base = wid * _BPW
    idx_bufs = (idx0_v, idx1_v)
    tr_bufs = idx_bufs
    isems = (isem0, isem1)
    gsems = (gsem0, gsem1)

    def start_idx(g, slot):
        off = (base + g * _CH) * _L
        pltpu.async_copy(idx_hbm.at[pl.ds(off, _IC)],
                         idx_bufs[slot], isems[slot])

    def wait_idx(slot):
        pltpu.make_async_copy(idx_hbm.at[pl.ds(0, _IC)],
                              idx_bufs[slot], isems[slot]).wait()

    def start_adds(slot):
        for l in range(_L):
            pltpu.async_copy(
                tab_hbm.at[tr_bufs[slot].at[pl.ds(l * _CH, _CH)]],
                acc_v.at[slot], gsems[slot], add=True)

    def wait_adds(slot):
        for l in range(_L):
            pltpu.make_async_copy(
                tab_hbm.at[tr_bufs[slot].at[pl.ds(l * _CH, _CH)]],
                acc_v.at[slot], gsems[slot]).wait()

    def zero_acc(slot):
        z = jnp.zeros((_LANES,), jnp.float32)

        def b_body(b, _):
            for c in range(_D // _LANES):
                acc_v[slot, b, pl.ds(c * _LANES, _LANES)] = z
            return 0

        lax.fori_loop(0, _CH, b_body, 0)

    def copy_out(g, slot):
        pltpu.sync_copy(acc_v.at[slot], out_hbm.at[pl.ds(base + g * _CH, _CH)])

    zero_acc(0)
    zero_acc(1)
    start_idx(0, 0)
    start_idx(1, 1)
    wait_idx(0)
    start_adds(0)

    def step(g, slot, last=False):
        other = 1 - slot
        wait_idx(other)
        start_adds(other)
        wait_adds(slot)
        if not last:
            start_idx(g + 2, slot)
        copy_out(g, slot)
        if not last:
            zero_acc(slot)

    def g2_body(g2, _):
        a = 2 * g2
        step(a, 0)
        step(a + 1, 1)
        return 0

    lax.fori_loop(0, _NCHUNK // 2 - 1, g2_body, 0)
    a = _NCHUNK - 2
    wait_idx(1)
    start_adds(1)
    wait_adds(0)
    copy_out(a, 0)
    wait_adds(1)
    copy_out(a + 1, 1)


_sc_pool = pl.kernel(
    _sc_pool_body,
    out_type=jax.ShapeDtypeStruct((_B, _D), jnp.float32),
    mesh=plsc.VectorSubcoreMesh(core_axis_name="c", subcore_axis_name="s"),
    scratch_types=[
        pltpu.VMEM((_IC,), jnp.int32),
        pltpu.VMEM((_IC,), jnp.int32),
        pltpu.VMEM((2, _CH, _D), jnp.float32),
        pltpu.SemaphoreType.DMA,
        pltpu.SemaphoreType.DMA,
        pltpu.SemaphoreType.DMA,
        pltpu.SemaphoreType.DMA,
    ],
)


_MBLK = 8192


def _one_tower(x, w0, b0, w1, b1, w2, b2):
    dn = (((1,), (1,)), ((), ()))
    h = jnp.maximum(lax.dot_general(
        x, w0, dn, preferred_element_type=jnp.float32) + b0, 0.0)
    h = jnp.maximum(lax.dot_general(
        h, w1, dn, preferred_element_type=jnp.float32) + b1, 0.0)
    return jnp.maximum(lax.dot_general(
        h, w2, dn, preferred_element_type=jnp.float32) + b2, 0.0)


def _mlp_body(qx_ref, cx_ref,
              qw0_ref, qb0_ref, qw1_ref, qb1_ref, qw2_ref, qb2_ref,
              cw0_ref, cb0_ref, cw1_ref, cb1_ref, cw2_ref, cb2_ref,
              qo_ref, co_ref):
    qo_ref[...] = _one_tower(qx_ref[...], qw0_ref[...], qb0_ref[...],
                             qw1_ref[...], qb1_ref[...],
                             qw2_ref[...], qb2_ref[...])
    co_ref[...] = _one_tower(cx_ref[...], cw0_ref[...], cb0_ref[...],
                             cw1_ref[...], cb1_ref[...],
                             cw2_ref[...], cb2_ref[...])


def _mlp_tower(pooled, ws):
    n0, n1, n2 = 128, 64, 32
    x_spec = pl.BlockSpec((_MBLK, _D), lambda i: (i, 0))
    full = lambda s: pl.BlockSpec(s, lambda i: tuple(0 for _ in s))
    w_specs = [full((n0, _D)), full((1, n0)),
               full((n1, n0)), full((1, n1)),
               full((n2, n1)), full((1, n2))]
    return pl.pallas_call(
        lambda x_ref, w0, b0, w1, b1, w2, b2, o_ref: o_ref.__setitem__(
            ..., _one_tower(x_ref[...], w0[...], b0[...], w1[...], b1[...],
                            w2[...], b2[...])),
        grid=(_B // _MBLK,),
        in_specs=[x_spec] + w_specs,
        out_specs=pl.BlockSpec((_MBLK, n2), lambda i: (i, 0)),
        out_shape=jax.ShapeDtypeStruct((_B, n2), jnp.float32),
    )(pooled, *ws)


def _permute_indices(idx):
    return (idx.astype(jnp.int32)
            .reshape(_NW, _NCHUNK, _CH, _L)
            .transpose(0, 1, 3, 2)
            .reshape(-1))


def kernel(query_indices, candidate_indices, q_table, c_table,
           q_w0, q_b0, q_w1, q_b1, q_w2, q_b2,
           c_w0, c_b0, c_w1, c_b1, c_w2, c_b2):
    qi = _permute_indices(query_indices)
    ci = _permute_indices(candidate_indices)
    qws = (q_w0, q_b0.reshape(1, -1), q_w1, q_b1.reshape(1, -1),
           q_w2, q_b2.reshape(1, -1))
    cws = (c_w0, c_b0.reshape(1, -1), c_w1, c_b1.reshape(1, -1),
           c_w2, c_b2.reshape(1, -1))
    q_pooled = _sc_pool(qi, q_table)
    c_pooled = _sc_pool(ci, c_table)
    query_embedding = _mlp_tower(q_pooled, qws)
    candidate_embedding = _mlp_tower(c_pooled, cws)
    return (query_embedding, candidate_embedding)
